# Initial kernel scaffold; baseline (speedup 1.0000x reference)
import jax
import jax.numpy as jnp
from jax.experimental import pallas as pl


def kernel(x, batch):
    raise NotImplementedError("write your pallas kernel here")


# speedup vs baseline: 4.5274x; 4.5274x over previous
import functools

import jax
import jax.numpy as jnp
from jax import lax
from jax.experimental import pallas as pl
from jax.experimental.pallas import tpu as pltpu
from jax.experimental.pallas import tpu_sc as plsc

N_ROWS = 320000
D = 128
N_SEG = 1024
NC = 2
NS = 16
NW = NC * NS
ROWS_PER_W = N_ROWS // NW
IDX_MINOR = 50
IDX_ROWS_PER_W = ROWS_PER_W // IDX_MINOR
CHUNK = 400
SUB = CHUNK // IDX_MINOR
N_CHUNKS = ROWS_PER_W // CHUNK
SEG_PER_TILE = N_SEG // NS
CNT_MINOR = 128


def _sc_body(x_hbm, b_hbm, sums_hbm, cnts_hbm,
             acc, cntacc, xbuf, idxall, ones, stage):
    c = lax.axis_index("c")
    s = lax.axis_index("s")
    wid = c * NS + s

    zeros16 = jnp.zeros((16,), jnp.float32)

    def _z_stage(k, _):
        i = k // (D // 16)
        j = k % (D // 16)
        stage[i, pl.ds(j * 16, 16)] = zeros16
        return 0
    lax.fori_loop(0, SEG_PER_TILE * (D // 16), _z_stage, 0)

    def _ones(k, _):
        i = k // (CNT_MINOR // 16)
        j = k % (CNT_MINOR // 16)
        ones[i, pl.ds(j * 16, 16)] = zeros16 + 1.0
        return 0
    lax.fori_loop(0, IDX_MINOR * (CNT_MINOR // 16), _ones, 0)

    seg0 = s * SEG_PER_TILE
    pltpu.sync_copy(stage, acc.at[pl.ds(seg0, SEG_PER_TILE)])
    pltpu.sync_copy(stage, cntacc.at[pl.ds(seg0, SEG_PER_TILE)])
    plsc.subcore_barrier()

    pltpu.sync_copy(b_hbm.at[pl.ds(wid * IDX_ROWS_PER_W, IDX_ROWS_PER_W)],
                    idxall)

    row0 = wid * ROWS_PER_W

    def _chunk(k, _):
        pltpu.sync_copy(x_hbm.at[pl.ds(row0 + k * CHUNK, CHUNK)], xbuf)
        for j in range(SUB):
            idx = idxall.at[k * SUB + j]
            pltpu.sync_copy(xbuf.at[pl.ds(j * IDX_MINOR, IDX_MINOR)],
                            acc.at[idx], add=True)
            pltpu.sync_copy(ones, cntacc.at[idx], add=True)
        return 0
    lax.fori_loop(0, N_CHUNKS, _chunk, 0)

    plsc.subcore_barrier()

    pltpu.sync_copy(acc.at[pl.ds(seg0, SEG_PER_TILE)], stage)
    pltpu.sync_copy(stage, sums_hbm.at[c, pl.ds(seg0, SEG_PER_TILE)])
    pltpu.sync_copy(cntacc.at[pl.ds(seg0, SEG_PER_TILE)], stage)
    pltpu.sync_copy(stage, cnts_hbm.at[c, pl.ds(seg0, SEG_PER_TILE)])


_sc_segment_sum = functools.partial(
    pl.kernel,
    out_type=(
        jax.ShapeDtypeStruct((NC, N_SEG, D), jnp.float32),
        jax.ShapeDtypeStruct((NC, N_SEG, CNT_MINOR), jnp.float32),
    ),
    mesh=plsc.VectorSubcoreMesh(core_axis_name="c", subcore_axis_name="s"),
    scratch_types=[
        pltpu.VMEM_SHARED((N_SEG, D), jnp.float32),
        pltpu.VMEM_SHARED((N_SEG, CNT_MINOR), jnp.float32),
        pltpu.VMEM((CHUNK, D), jnp.float32),
        pltpu.VMEM((IDX_ROWS_PER_W, IDX_MINOR), jnp.int32),
        pltpu.VMEM((IDX_MINOR, CNT_MINOR), jnp.float32),
        pltpu.VMEM((SEG_PER_TILE, D), jnp.float32),
    ],
)(_sc_body)


def _combine_body(s_ref, c_ref, o_ref):
    sm = s_ref[...]
    cn = c_ref[...]
    tot = sm[0] + sm[1]
    cnt = jnp.maximum(cn[0] + cn[1], 1.0)
    o_ref[...] = tot / cnt[:, None]


def _combine(sums, counts):
    blk = 128
    return pl.pallas_call(
        _combine_body,
        grid=(N_SEG // blk,),
        in_specs=[
            pl.BlockSpec((NC, blk, D), lambda i: (0, i, 0)),
            pl.BlockSpec((NC, blk), lambda i: (0, i)),
        ],
        out_specs=pl.BlockSpec((blk, D), lambda i: (i, 0)),
        out_shape=jax.ShapeDtypeStruct((N_SEG, D), jnp.float32),
    )(sums, counts)


def kernel(x, batch):
    batch2d = batch.astype(jnp.int32).reshape(N_ROWS // IDX_MINOR, IDX_MINOR)
    sums, cnts = _sc_segment_sum(x, batch2d)
    return _combine(sums, cnts[:, :, 0])


# speedup vs baseline: 6.3013x; 1.3918x over previous
import functools

import jax
import jax.numpy as jnp
from jax import lax
from jax.experimental import pallas as pl
from jax.experimental.pallas import tpu as pltpu
from jax.experimental.pallas import tpu_sc as plsc

N_ROWS = 320000
D = 128
N_SEG = 1024
NC = 2
NS = 16
NW = NC * NS
ROWS_PER_W = N_ROWS // NW
IDX_MINOR = 50
IDX_ROWS_PER_W = ROWS_PER_W // IDX_MINOR
CHUNK = 400
SUB = CHUNK // IDX_MINOR
N_CHUNKS = ROWS_PER_W // CHUNK
SEG_PER_TILE = N_SEG // NS
HROWS = N_SEG // 8
HSEG_PER_TILE = HROWS // NS


def _sc_body(x_hbm, b_hbm, sums_hbm, cnts_hbm,
             acc, cntsq, xbuf, idxall, hist, idbuf, czero, stage):
    c = lax.axis_index("c")
    s = lax.axis_index("s")
    wid = c * NS + s

    zeros16 = jnp.zeros((16,), jnp.float32)
    zeros16i = jnp.zeros((16,), jnp.int32)
    iota16 = lax.iota(jnp.int32, 16)

    def _z_stage(k, _):
        stage[k // 8, pl.ds((k % 8) * 16, 16)] = zeros16
        return 0
    lax.fori_loop(0, SEG_PER_TILE * 8, _z_stage, 0)

    def _z_hist(k, _):
        hist[k // 8, pl.ds((k % 8) * 16, 16)] = zeros16i
        return 0
    lax.fori_loop(0, HROWS * 8, _z_hist, 0)

    def _z_czero(k, _):
        czero[k // 8, pl.ds((k % 8) * 16, 16)] = zeros16i
        return 0
    lax.fori_loop(0, HSEG_PER_TILE * 8, _z_czero, 0)

    for j in range(8):
        idbuf[0, pl.ds(j * 16, 16)] = iota16 + (j * 16)

    seg0 = s * SEG_PER_TILE
    hseg0 = s * HSEG_PER_TILE
    pltpu.sync_copy(stage, acc.at[pl.ds(seg0, SEG_PER_TILE)])
    pltpu.sync_copy(czero, cntsq.at[pl.ds(hseg0, HSEG_PER_TILE)])
    plsc.subcore_barrier()

    pltpu.sync_copy(b_hbm.at[pl.ds(wid * IDX_ROWS_PER_W, IDX_ROWS_PER_W)],
                    idxall)

    ones16 = zeros16i + 1
    tailmask = iota16 >= 14

    def _hist_row(r, _):
        for (off, msk) in ((0, None), (16, None), (32, None), (34, tailmask)):
            v = idxall[r, pl.ds(off, 16)]
            plsc.addupdate_scatter(hist, [v >> 3, (v & 7) * 16], ones16,
                                   mask=msk)
        return 0
    lax.fori_loop(0, IDX_ROWS_PER_W, _hist_row, 0)

    row0 = wid * ROWS_PER_W

    def _chunk(k, _):
        pltpu.sync_copy(x_hbm.at[pl.ds(row0 + k * CHUNK, CHUNK)], xbuf)
        for j in range(SUB):
            idx = idxall.at[k * SUB + j]
            pltpu.sync_copy(xbuf.at[pl.ds(j * IDX_MINOR, IDX_MINOR)],
                            acc.at[idx], add=True)
        return 0
    lax.fori_loop(0, N_CHUNKS, _chunk, 0)

    pltpu.sync_copy(hist, cntsq.at[idbuf.at[0]], add=True)

    plsc.subcore_barrier()

    pltpu.sync_copy(acc.at[pl.ds(seg0, SEG_PER_TILE)], stage)
    pltpu.sync_copy(stage, sums_hbm.at[c, pl.ds(seg0, SEG_PER_TILE)])
    pltpu.sync_copy(cntsq.at[pl.ds(hseg0, HSEG_PER_TILE)], czero)
    pltpu.sync_copy(czero, cnts_hbm.at[c, pl.ds(hseg0, HSEG_PER_TILE)])


_sc_segment_sum = functools.partial(
    pl.kernel,
    out_type=(
        jax.ShapeDtypeStruct((NC, N_SEG, D), jnp.float32),
        jax.ShapeDtypeStruct((NC, HROWS, 128), jnp.int32),
    ),
    mesh=plsc.VectorSubcoreMesh(core_axis_name="c", subcore_axis_name="s"),
    scratch_types=[
        pltpu.VMEM_SHARED((N_SEG, D), jnp.float32),
        pltpu.VMEM_SHARED((HROWS, 128), jnp.int32),
        pltpu.VMEM((CHUNK, D), jnp.float32),
        pltpu.VMEM((IDX_ROWS_PER_W, IDX_MINOR), jnp.int32),
        pltpu.VMEM((HROWS, 128), jnp.int32),
        pltpu.VMEM((1, 128), jnp.int32),
        pltpu.VMEM((HSEG_PER_TILE, 128), jnp.int32),
        pltpu.VMEM((SEG_PER_TILE, D), jnp.float32),
    ],
    compiler_params=pltpu.CompilerParams(needs_layout_passes=False),
)(_sc_body)


def _combine_body(s_ref, c_ref, o_ref):
    sm = s_ref[...]
    cn = c_ref[...]
    tot = sm[0] + sm[1]
    cnt = jnp.maximum((cn[0] + cn[1]).astype(jnp.float32), 1.0)
    o_ref[...] = tot / cnt[:, None]


def _combine(sums, counts):
    blk = 128
    return pl.pallas_call(
        _combine_body,
        grid=(N_SEG // blk,),
        in_specs=[
            pl.BlockSpec((NC, blk, D), lambda i: (0, i, 0)),
            pl.BlockSpec((NC, blk), lambda i: (0, i)),
        ],
        out_specs=pl.BlockSpec((blk, D), lambda i: (i, 0)),
        out_shape=jax.ShapeDtypeStruct((N_SEG, D), jnp.float32),
    )(sums, counts)


def kernel(x, batch):
    batch2d = batch.astype(jnp.int32).reshape(N_ROWS // IDX_MINOR, IDX_MINOR)
    sums, cnts = _sc_segment_sum(x, batch2d)
    counts = cnts[:, :, ::16].reshape(NC, N_SEG)
    return _combine(sums, counts)


# speedup vs baseline: 8.0773x; 1.2818x over previous
import functools

import jax
import jax.numpy as jnp
from jax import lax
from jax.experimental import pallas as pl
from jax.experimental.pallas import tpu as pltpu
from jax.experimental.pallas import tpu_sc as plsc

N_ROWS = 320000
D = 128
N_SEG = 1024
NC = 2
NS = 16
NW = NC * NS
ROWS_PER_W = N_ROWS // NW
IDX_MINOR = 50
IDX_ROWS_PER_W = ROWS_PER_W // IDX_MINOR
CHUNK = 200
SUB = CHUNK // IDX_MINOR
N_CHUNKS = ROWS_PER_W // CHUNK
SEG_PER_TILE = N_SEG // NS
HROWS = N_SEG // 8
HSEG_PER_TILE = HROWS // NS


def _sc_body(x_hbm, b_hbm, sums_hbm, cnts_hbm,
             acc, cntsq, xbuf0, xbuf1, idxall, hist, idbuf, fzero,
             gsem0, gsem1, ssem):
    c = lax.axis_index("c")
    s = lax.axis_index("s")
    wid = c * NS + s

    zeros16 = jnp.zeros((16,), jnp.float32)
    zeros16i = jnp.zeros((16,), jnp.int32)
    iota16 = lax.iota(jnp.int32, 16)

    def _z_hist(k, _):
        hist[k // 8, pl.ds((k % 8) * 16, 16)] = zeros16i
        return 0
    lax.fori_loop(0, HROWS * 8, _z_hist, 0)

    def _z_fzero(k, _):
        fzero[k // 8, pl.ds((k % 8) * 16, 16)] = zeros16
        return 0
    lax.fori_loop(0, HSEG_PER_TILE * 8, _z_fzero, 0)

    for j in range(8):
        idbuf[0, pl.ds(j * 16, 16)] = iota16 + (j * 16)

    seg0 = s * SEG_PER_TILE
    hseg0 = s * HSEG_PER_TILE
    for j in range(SEG_PER_TILE // HSEG_PER_TILE):
        pltpu.sync_copy(fzero,
                        acc.at[pl.ds(seg0 + j * HSEG_PER_TILE,
                                     HSEG_PER_TILE)])
    pltpu.sync_copy(hist.at[pl.ds(0, HSEG_PER_TILE)],
                    cntsq.at[pl.ds(hseg0, HSEG_PER_TILE)])
    plsc.subcore_barrier()

    pltpu.sync_copy(b_hbm.at[pl.ds(wid * IDX_ROWS_PER_W, IDX_ROWS_PER_W)],
                    idxall)

    row0 = wid * ROWS_PER_W
    bufs = (xbuf0, xbuf1)
    gsems = (gsem0, gsem1)

    def _gather(k, b):
        pltpu.async_copy(
            x_hbm.at[pl.ds(row0 + k * CHUNK, CHUNK)], bufs[b], gsems[b])

    def _drain_gather(b):
        pltpu.make_async_copy(x_hbm.at[pl.ds(0, CHUNK)], bufs[b],
                              gsems[b]).wait()

    def _issue_scatters(k, b):
        for j in range(SUB):
            idx = idxall.at[k * SUB + j]
            pltpu.async_copy(bufs[b].at[pl.ds(j * IDX_MINOR, IDX_MINOR)],
                             acc.at[idx], ssem, add=True)

    def _drain_scatters(b):
        pltpu.make_async_copy(x_hbm.at[pl.ds(0, CHUNK)], bufs[b],
                              ssem).wait()

    _gather(0, 0)
    _gather(1, 1)

    ones16 = zeros16i + 1
    tailmask = iota16 >= 14

    def _hist_row(r, _):
        for (off, msk) in ((0, None), (16, None), (32, None), (34, tailmask)):
            v = idxall[r, pl.ds(off, 16)]
            plsc.addupdate_scatter(hist, [v >> 3, (v & 7) * 16], ones16,
                                   mask=msk)
        return 0
    lax.fori_loop(0, IDX_ROWS_PER_W, _hist_row, 0)

    def _pair(g, _):
        k0 = 2 * g
        _drain_gather(0)
        _issue_scatters(k0, 0)
        _drain_gather(1)
        _drain_scatters(0)
        _gather(k0 + 2, 0)
        _issue_scatters(k0 + 1, 1)
        _drain_scatters(1)
        _gather(k0 + 3, 1)
        return 0
    lax.fori_loop(0, N_CHUNKS // 2 - 1, _pair, 0)

    _drain_gather(0)
    _issue_scatters(N_CHUNKS - 2, 0)
    _drain_gather(1)
    _drain_scatters(0)
    _issue_scatters(N_CHUNKS - 1, 1)
    _drain_scatters(1)

    pltpu.sync_copy(hist, cntsq.at[idbuf.at[0]], add=True)

    plsc.subcore_barrier()

    pltpu.sync_copy(acc.at[pl.ds(seg0, SEG_PER_TILE)],
                    sums_hbm.at[c, pl.ds(seg0, SEG_PER_TILE)])
    pltpu.sync_copy(cntsq.at[pl.ds(hseg0, HSEG_PER_TILE)],
                    cnts_hbm.at[c, pl.ds(hseg0, HSEG_PER_TILE)])


_sc_segment_sum = functools.partial(
    pl.kernel,
    out_type=(
        jax.ShapeDtypeStruct((NC, N_SEG, D), jnp.float32),
        jax.ShapeDtypeStruct((NC, HROWS, 128), jnp.int32),
    ),
    mesh=plsc.VectorSubcoreMesh(core_axis_name="c", subcore_axis_name="s"),
    scratch_types=[
        pltpu.VMEM_SHARED((N_SEG, D), jnp.float32),
        pltpu.VMEM_SHARED((HROWS, 128), jnp.int32),
        pltpu.VMEM((CHUNK, D), jnp.float32),
        pltpu.VMEM((CHUNK, D), jnp.float32),
        pltpu.VMEM((IDX_ROWS_PER_W, IDX_MINOR), jnp.int32),
        pltpu.VMEM((HROWS, 128), jnp.int32),
        pltpu.VMEM((1, 128), jnp.int32),
        pltpu.VMEM((HSEG_PER_TILE, 128), jnp.float32),
        pltpu.SemaphoreType.DMA,
        pltpu.SemaphoreType.DMA,
        pltpu.SemaphoreType.DMA,
    ],
    compiler_params=pltpu.CompilerParams(needs_layout_passes=False),
)(_sc_body)


def _combine_body(s_ref, c_ref, o_ref):
    sm = s_ref[...]
    cn = c_ref[...]
    tot = sm[0] + sm[1]
    cnt = jnp.maximum((cn[0] + cn[1]).astype(jnp.float32), 1.0)
    o_ref[...] = tot / cnt[:, None]


def _combine(sums, counts):
    blk = 128
    return pl.pallas_call(
        _combine_body,
        grid=(N_SEG // blk,),
        in_specs=[
            pl.BlockSpec((NC, blk, D), lambda i: (0, i, 0)),
            pl.BlockSpec((NC, blk), lambda i: (0, i)),
        ],
        out_specs=pl.BlockSpec((blk, D), lambda i: (i, 0)),
        out_shape=jax.ShapeDtypeStruct((N_SEG, D), jnp.float32),
    )(sums, counts)


def kernel(x, batch):
    batch2d = batch.astype(jnp.int32).reshape(N_ROWS // IDX_MINOR, IDX_MINOR)
    sums, cnts = _sc_segment_sum(x, batch2d)
    counts = cnts[:, :, ::16].reshape(NC, N_SEG)
    return _combine(sums, counts)


# speedup vs baseline: 8.6592x; 1.0720x over previous
import functools

import jax
import jax.numpy as jnp
from jax import lax
from jax.experimental import pallas as pl
from jax.experimental.pallas import tpu as pltpu
from jax.experimental.pallas import tpu_sc as plsc

N_ROWS = 320000
D = 128
N_SEG = 1024
NC = 2
NS = 16
NW = NC * NS
GROUP = 128
N_GROUPS = N_ROWS // GROUP
GROUPS_PER_W = N_GROUPS // NW
N_EXTRA = N_GROUPS - GROUPS_PER_W * NW
IDX_PAD_ROWS = 2504
IDX_WIN = 88
SEG_PER_TILE = N_SEG // NS
HROWS = N_SEG // 8
HSEG_PER_TILE = HROWS // NS
N_PAIRS = GROUPS_PER_W // 2


def _sc_body(x_hbm, b_hbm, sums_hbm, cnts_hbm,
             acc, cntsq, xbuf0, xbuf1, idxall, hist, idbuf, fzero,
             gsem0, gsem1, ssem):
    c = lax.axis_index("c")
    s = lax.axis_index("s")
    wid = c * NS + s

    zeros16 = jnp.zeros((16,), jnp.float32)
    zeros16i = jnp.zeros((16,), jnp.int32)
    iota16 = lax.iota(jnp.int32, 16)

    def _z_hist(k, _):
        hist[k // 8, pl.ds((k % 8) * 16, 16)] = zeros16i
        return 0
    lax.fori_loop(0, HROWS * 8, _z_hist, 0)

    def _z_fzero(k, _):
        fzero[k // 8, pl.ds((k % 8) * 16, 16)] = zeros16
        return 0
    lax.fori_loop(0, HSEG_PER_TILE * 8, _z_fzero, 0)

    for j in range(8):
        idbuf[0, pl.ds(j * 16, 16)] = iota16 + (j * 16)

    seg0 = s * SEG_PER_TILE
    hseg0 = s * HSEG_PER_TILE
    for j in range(SEG_PER_TILE // HSEG_PER_TILE):
        pltpu.sync_copy(fzero,
                        acc.at[pl.ds(seg0 + j * HSEG_PER_TILE,
                                     HSEG_PER_TILE)])
    pltpu.sync_copy(hist.at[pl.ds(0, HSEG_PER_TILE)],
                    cntsq.at[pl.ds(hseg0, HSEG_PER_TILE)])
    plsc.subcore_barrier()

    start = GROUPS_PER_W * wid + jnp.minimum(wid, N_EXTRA)
    has_extra = wid < N_EXTRA
    off = start & 7
    wstart = pl.multiple_of(start - off, 8)

    pltpu.sync_copy(b_hbm.at[pl.ds(wstart, IDX_WIN)], idxall)

    bufs = (xbuf0, xbuf1)
    gsems = (gsem0, gsem1)

    def _gather(k, b):
        pltpu.async_copy(
            x_hbm.at[pl.ds((start + k) * GROUP, GROUP)], bufs[b], gsems[b])

    def _drain_gather(b):
        pltpu.make_async_copy(x_hbm.at[pl.ds(0, GROUP)], bufs[b],
                              gsems[b]).wait()

    def _scatter(k, b):
        pltpu.async_copy(bufs[b], acc.at[idxall.at[off + k]], ssem,
                         add=True)

    def _drain_scatter(b):
        pltpu.make_async_copy(x_hbm.at[pl.ds(0, GROUP)], bufs[b],
                              ssem).wait()

    _gather(0, 0)
    _gather(1, 1)

    ones16 = zeros16i + 1
    ngroups = GROUPS_PER_W + has_extra.astype(jnp.int32)

    def _hist_row(r, _):
        for j in range(8):
            v = idxall[off + r, pl.ds(j * 16, 16)]
            plsc.addupdate_scatter(hist, [v >> 3, (v & 7) * 16], ones16)
        return 0
    lax.fori_loop(0, ngroups, _hist_row, 0)

    def _pair(g, _):
        k0 = 2 * g
        _drain_gather(0)
        _scatter(k0, 0)
        _drain_gather(1)
        _drain_scatter(0)
        _gather(k0 + 2, 0)
        _scatter(k0 + 1, 1)
        _drain_scatter(1)
        _gather(k0 + 3, 1)
        return 0
    lax.fori_loop(0, N_PAIRS - 1, _pair, 0)

    kl = 2 * (N_PAIRS - 1)
    _drain_gather(0)
    _scatter(kl, 0)
    _drain_gather(1)
    _drain_scatter(0)

    @pl.when(has_extra)
    def _extra_gather():
        _gather(GROUPS_PER_W, 0)

    _scatter(kl + 1, 1)
    _drain_scatter(1)

    @pl.when(has_extra)
    def _extra_scatter():
        _drain_gather(0)
        _scatter(GROUPS_PER_W, 0)
        _drain_scatter(0)

    pltpu.sync_copy(hist, cntsq.at[idbuf.at[0]], add=True)

    plsc.subcore_barrier()

    pltpu.sync_copy(acc.at[pl.ds(seg0, SEG_PER_TILE)],
                    sums_hbm.at[c, pl.ds(seg0, SEG_PER_TILE)])
    pltpu.sync_copy(cntsq.at[pl.ds(hseg0, HSEG_PER_TILE)],
                    cnts_hbm.at[c, pl.ds(hseg0, HSEG_PER_TILE)])


_sc_segment_sum = functools.partial(
    pl.kernel,
    out_type=(
        jax.ShapeDtypeStruct((NC, N_SEG, D), jnp.float32),
        jax.ShapeDtypeStruct((NC, HROWS, 128), jnp.int32),
    ),
    mesh=plsc.VectorSubcoreMesh(core_axis_name="c", subcore_axis_name="s"),
    scratch_types=[
        pltpu.VMEM_SHARED((N_SEG, D), jnp.float32),
        pltpu.VMEM_SHARED((HROWS, 128), jnp.int32),
        pltpu.VMEM((GROUP, D), jnp.float32),
        pltpu.VMEM((GROUP, D), jnp.float32),
        pltpu.VMEM((IDX_WIN, 128), jnp.int32),
        pltpu.VMEM((HROWS, 128), jnp.int32),
        pltpu.VMEM((1, 128), jnp.int32),
        pltpu.VMEM((HSEG_PER_TILE, 128), jnp.float32),
        pltpu.SemaphoreType.DMA,
        pltpu.SemaphoreType.DMA,
        pltpu.SemaphoreType.DMA,
    ],
    compiler_params=pltpu.CompilerParams(needs_layout_passes=False),
)(_sc_body)


def _combine_body(s_ref, c_ref, o_ref):
    sm = s_ref[...]
    cn = c_ref[...]
    tot = sm[0] + sm[1]
    cnt = jnp.maximum((cn[0] + cn[1]).astype(jnp.float32), 1.0)
    o_ref[...] = tot / cnt[:, None]


def _combine(sums, counts):
    blk = 128
    return pl.pallas_call(
        _combine_body,
        grid=(N_SEG // blk,),
        in_specs=[
            pl.BlockSpec((NC, blk, D), lambda i: (0, i, 0)),
            pl.BlockSpec((NC, blk), lambda i: (0, i)),
        ],
        out_specs=pl.BlockSpec((blk, D), lambda i: (i, 0)),
        out_shape=jax.ShapeDtypeStruct((N_SEG, D), jnp.float32),
    )(sums, counts)


def kernel(x, batch):
    b32 = batch.astype(jnp.int32)
    pad = jnp.zeros((IDX_PAD_ROWS * GROUP - N_ROWS,), jnp.int32)
    batch2d = jnp.concatenate([b32, pad]).reshape(IDX_PAD_ROWS, GROUP)
    sums, cnts = _sc_segment_sum(x, batch2d)
    counts = cnts[:, :, ::16].reshape(NC, N_SEG)
    return _combine(sums, counts)


# speedup vs baseline: 9.1248x; 1.0538x over previous
import functools

import jax
import jax.numpy as jnp
from jax import lax
from jax.experimental import pallas as pl
from jax.experimental.pallas import tpu as pltpu
from jax.experimental.pallas import tpu_sc as plsc

N_ROWS = 320000
D = 128
N_SEG = 1024
NC = 2
NS = 16
NW = NC * NS
GROUP = 128
N_GROUPS = N_ROWS // GROUP
GROUPS_PER_W = N_GROUPS // NW
N_EXTRA = N_GROUPS - GROUPS_PER_W * NW
IDX_PAD_ROWS = 2504
IDX_WIN = 88
SEG_PER_TILE = N_SEG // NS
HROWS = N_SEG // 8
HSEG_PER_TILE = HROWS // NS
N_PAIRS = GROUPS_PER_W // 2


def _sc_body(x_hbm, b_hbm, sums_hbm, cnts_hbm,
             acc, cntsq, xbuf0, xbuf1, idxall, hist, idbuf, fzero,
             gsem0, gsem1, ssem):
    c = lax.axis_index("c")
    s = lax.axis_index("s")
    wid = c * NS + s

    zeros16 = jnp.zeros((16,), jnp.float32)
    zeros16i = jnp.zeros((16,), jnp.int32)
    iota16 = lax.iota(jnp.int32, 16)

    def _z_hist(k, _):
        hist[k // 8, pl.ds((k % 8) * 16, 16)] = zeros16i
        return 0
    lax.fori_loop(0, HROWS * 8, _z_hist, 0)

    def _z_fzero(k, _):
        fzero[k // 8, pl.ds((k % 8) * 16, 16)] = zeros16
        return 0
    lax.fori_loop(0, HSEG_PER_TILE * 8, _z_fzero, 0)

    for j in range(8):
        idbuf[0, pl.ds(j * 16, 16)] = iota16 + (j * 16)

    seg0 = s * SEG_PER_TILE
    hseg0 = s * HSEG_PER_TILE
    for j in range(SEG_PER_TILE // HSEG_PER_TILE):
        pltpu.sync_copy(fzero,
                        acc.at[pl.ds(seg0 + j * HSEG_PER_TILE,
                                     HSEG_PER_TILE)])
    pltpu.sync_copy(hist.at[pl.ds(0, HSEG_PER_TILE)],
                    cntsq.at[pl.ds(hseg0, HSEG_PER_TILE)])
    plsc.subcore_barrier()

    start = GROUPS_PER_W * wid + jnp.minimum(wid, N_EXTRA)
    has_extra = wid < N_EXTRA
    off = start & 7
    wstart = pl.multiple_of(start - off, 8)

    pltpu.sync_copy(b_hbm.at[pl.ds(wstart, IDX_WIN)], idxall)

    bufs = (xbuf0, xbuf1)
    gsems = (gsem0, gsem1)

    def _gather(k, b):
        pltpu.async_copy(
            x_hbm.at[pl.ds((start + k) * GROUP, GROUP)], bufs[b], gsems[b])

    def _drain_gather(b):
        pltpu.make_async_copy(x_hbm.at[pl.ds(0, GROUP)], bufs[b],
                              gsems[b]).wait()

    def _scatter(k, b):
        pltpu.async_copy(bufs[b], acc.at[idxall.at[off + k]], ssem,
                         add=True)

    def _drain_scatter(b):
        pltpu.make_async_copy(x_hbm.at[pl.ds(0, GROUP)], bufs[b],
                              ssem).wait()

    _gather(0, 0)
    _gather(1, 1)

    ones16 = zeros16i + 1

    def _hist_row(r):
        for j in range(8):
            v = idxall[off + r, pl.ds(j * 16, 16)]
            plsc.addupdate_scatter(hist, [v >> 3, (v & 7) * 16], ones16)

    def _pair(g, _):
        k0 = 2 * g
        _hist_row(k0)
        _drain_gather(0)
        _scatter(k0, 0)
        _hist_row(k0 + 1)
        _drain_gather(1)
        _drain_scatter(0)
        _gather(k0 + 2, 0)
        _scatter(k0 + 1, 1)
        _drain_scatter(1)
        _gather(k0 + 3, 1)
        return 0
    lax.fori_loop(0, N_PAIRS - 1, _pair, 0)

    kl = 2 * (N_PAIRS - 1)
    _hist_row(kl)
    _drain_gather(0)
    _scatter(kl, 0)
    _hist_row(kl + 1)
    _drain_gather(1)
    _drain_scatter(0)

    @pl.when(has_extra)
    def _extra_gather():
        _gather(GROUPS_PER_W, 0)

    _scatter(kl + 1, 1)
    _drain_scatter(1)

    @pl.when(has_extra)
    def _extra_scatter():
        _hist_row(GROUPS_PER_W)
        _drain_gather(0)
        _scatter(GROUPS_PER_W, 0)
        _drain_scatter(0)

    pltpu.sync_copy(hist, cntsq.at[idbuf.at[0]], add=True)

    plsc.subcore_barrier()

    pltpu.sync_copy(acc.at[pl.ds(seg0, SEG_PER_TILE)],
                    sums_hbm.at[c, pl.ds(seg0, SEG_PER_TILE)])
    pltpu.sync_copy(cntsq.at[pl.ds(hseg0, HSEG_PER_TILE)],
                    cnts_hbm.at[c, pl.ds(hseg0, HSEG_PER_TILE)])


_sc_segment_sum = functools.partial(
    pl.kernel,
    out_type=(
        jax.ShapeDtypeStruct((NC, N_SEG, D), jnp.float32),
        jax.ShapeDtypeStruct((NC, HROWS, 128), jnp.int32),
    ),
    mesh=plsc.VectorSubcoreMesh(core_axis_name="c", subcore_axis_name="s"),
    scratch_types=[
        pltpu.VMEM_SHARED((N_SEG, D), jnp.float32),
        pltpu.VMEM_SHARED((HROWS, 128), jnp.int32),
        pltpu.VMEM((GROUP, D), jnp.float32),
        pltpu.VMEM((GROUP, D), jnp.float32),
        pltpu.VMEM((IDX_WIN, 128), jnp.int32),
        pltpu.VMEM((HROWS, 128), jnp.int32),
        pltpu.VMEM((1, 128), jnp.int32),
        pltpu.VMEM((HSEG_PER_TILE, 128), jnp.float32),
        pltpu.SemaphoreType.DMA,
        pltpu.SemaphoreType.DMA,
        pltpu.SemaphoreType.DMA,
    ],
    compiler_params=pltpu.CompilerParams(needs_layout_passes=False),
)(_sc_body)


def _combine_body(s_ref, c_ref, o_ref):
    sm = s_ref[...]
    cn = c_ref[...]
    tot = sm[0] + sm[1]
    cnt = jnp.maximum((cn[0] + cn[1]).astype(jnp.float32), 1.0)
    o_ref[...] = tot / cnt[:, None]


def _combine(sums, counts):
    return pl.pallas_call(
        _combine_body,
        out_shape=jax.ShapeDtypeStruct((N_SEG, D), jnp.float32),
    )(sums, counts)


def kernel(x, batch):
    b32 = batch.astype(jnp.int32)
    pad = jnp.zeros((IDX_PAD_ROWS * GROUP - N_ROWS,), jnp.int32)
    batch2d = jnp.concatenate([b32, pad]).reshape(IDX_PAD_ROWS, GROUP)
    sums, cnts = _sc_segment_sum(x, batch2d)
    counts = cnts[:, :, ::16].reshape(NC, N_SEG)
    return _combine(sums, counts)
